# Initial kernel scaffold; baseline (speedup 1.0000x reference)
import jax
import jax.numpy as jnp
from jax.experimental import pallas as pl


def kernel(X, table):
    raise NotImplementedError("write your pallas kernel here")


# speedup vs baseline: 1.9398x; 1.9398x over previous
import functools

import jax
import jax.numpy as jnp
from jax import lax
from jax.experimental import pallas as pl
from jax.experimental.pallas import tpu as pltpu
from jax.experimental.pallas import tpu_sc as plsc

BATCH = 16384
NUM_FIELDS = 26
VOCAB = 1000000
EMBED_DIM = 64
LANES = 16
SLICES = EMBED_DIM // LANES

ITEMS = BATCH * NUM_FIELDS * SLICES
IDX_COLS = 128
IDX_ROWS = ITEMS // IDX_COLS

NUM_CORES = 2
NUM_SUBCORES = 16
NUM_WORKERS = NUM_CORES * NUM_SUBCORES
ROWS_PER_WORKER = IDX_ROWS // NUM_WORKERS
CHUNK_ROWS = 16
CHUNKS = ROWS_PER_WORKER // CHUNK_ROWS

_mesh = plsc.VectorSubcoreMesh(core_axis_name="c", subcore_axis_name="s")


@functools.partial(
    pl.kernel,
    mesh=_mesh,
    compiler_params=pltpu.CompilerParams(use_tc_tiling_on_sc=False),
    out_type=jax.ShapeDtypeStruct((IDX_ROWS, IDX_COLS, LANES), jnp.float32),
    scratch_types=[
        pltpu.VMEM((CHUNK_ROWS, IDX_COLS), jnp.int32),
        pltpu.VMEM((CHUNK_ROWS, IDX_COLS, LANES), jnp.float32),
        pltpu.SemaphoreType.DMA,
    ],
)
def _mh_embed_gather(idx_hbm, table_hbm, out_hbm, idx_v, buf_v, sem):
    wid = lax.axis_index("s") * NUM_CORES + lax.axis_index("c")
    base = wid * ROWS_PER_WORKER

    def chunk_body(g, carry):
        off = base + g * CHUNK_ROWS
        pltpu.sync_copy(idx_hbm.at[pl.ds(off, CHUNK_ROWS)], idx_v)
        copies = [
            pltpu.async_copy(table_hbm.at[idx_v.at[j]], buf_v.at[j], sem)
            for j in range(CHUNK_ROWS)
        ]
        for c in copies:
            c.wait()
        pltpu.sync_copy(buf_v, out_hbm.at[pl.ds(off, CHUNK_ROWS)])
        return carry

    lax.fori_loop(0, CHUNKS, chunk_body, 0)


def kernel(X, table):
    ks = jnp.arange(SLICES, dtype=jnp.int32)
    idx = (X.astype(jnp.int32)[:, None, :] * SLICES + ks[None, :, None])
    idx = idx.reshape(IDX_ROWS, IDX_COLS)
    table16 = table.reshape(VOCAB * SLICES, LANES)
    out = _mh_embed_gather(idx, table16)
    return out.reshape(BATCH, NUM_FIELDS * EMBED_DIM)


# speedup vs baseline: 1.9981x; 1.0300x over previous
import functools

import jax
import jax.numpy as jnp
from jax import lax
from jax.experimental import pallas as pl
from jax.experimental.pallas import tpu as pltpu
from jax.experimental.pallas import tpu_sc as plsc

BATCH = 16384
NUM_FIELDS = 26
VOCAB = 1000000
EMBED_DIM = 64
LANES = 16
SLICES = EMBED_DIM // LANES

ITEMS = BATCH * NUM_FIELDS * SLICES
IDX_COLS = 128
IDX_ROWS = ITEMS // IDX_COLS

NUM_CORES = 2
NUM_SUBCORES = 16
NUM_WORKERS = NUM_CORES * NUM_SUBCORES
ROWS_PER_WORKER = IDX_ROWS // NUM_WORKERS
CHUNK_ROWS = 16
CHUNKS = ROWS_PER_WORKER // CHUNK_ROWS

_mesh = plsc.VectorSubcoreMesh(core_axis_name="c", subcore_axis_name="s")


@functools.partial(
    pl.kernel,
    mesh=_mesh,
    compiler_params=pltpu.CompilerParams(use_tc_tiling_on_sc=False),
    out_type=jax.ShapeDtypeStruct((IDX_ROWS, IDX_COLS, LANES), jnp.float32),
    scratch_types=[
        pltpu.VMEM((2, CHUNK_ROWS, IDX_COLS), jnp.int32),
        pltpu.VMEM((2, CHUNK_ROWS, IDX_COLS, LANES), jnp.float32),
        pltpu.SemaphoreType.DMA,
        pltpu.SemaphoreType.DMA,
        pltpu.SemaphoreType.DMA,
        pltpu.SemaphoreType.DMA,
    ],
)
def _mh_embed_gather(idx_hbm, table_hbm, out_hbm, idx_v, buf_v,
                     gsem0, gsem1, osem0, osem1):
    wid = lax.axis_index("s") * NUM_CORES + lax.axis_index("c")
    base = wid * ROWS_PER_WORKER
    gsems = (gsem0, gsem1)
    osems = (osem0, osem1)

    def fire(g, slot):
        off = base + g * CHUNK_ROWS
        pltpu.sync_copy(idx_hbm.at[pl.ds(off, CHUNK_ROWS)], idx_v.at[slot])
        for j in range(CHUNK_ROWS):
            pltpu.async_copy(
                table_hbm.at[idx_v.at[slot, j]], buf_v.at[slot, j], gsems[slot])

    def wait_gathers(slot):
        for j in range(CHUNK_ROWS):
            pltpu.make_async_copy(
                table_hbm.at[idx_v.at[slot, j]], buf_v.at[slot, j],
                gsems[slot]).wait()

    def start_write(g, slot):
        off = base + g * CHUNK_ROWS
        pltpu.async_copy(
            buf_v.at[slot], out_hbm.at[pl.ds(off, CHUNK_ROWS)], osems[slot])

    def wait_write(slot):
        pltpu.make_async_copy(
            buf_v.at[slot], out_hbm.at[pl.ds(0, CHUNK_ROWS)], osems[slot]
        ).wait()

    def pair_body(i, carry):
        for b in range(2):
            g = i * 2 + b
            slot = b
            @pl.when(g >= 2)
            def _():
                wait_write(slot)

            fire(g, slot)
            wait_gathers(slot)
            start_write(g, slot)
        return carry

    lax.fori_loop(0, CHUNKS // 2, pair_body, 0)
    wait_write(0)
    wait_write(1)


def kernel(X, table):
    ks = jnp.arange(SLICES, dtype=jnp.int32)
    idx = (X.astype(jnp.int32)[:, None, :] * SLICES + ks[None, :, None])
    idx = idx.reshape(IDX_ROWS, IDX_COLS)
    table16 = table.reshape(VOCAB * SLICES, LANES)
    out = _mh_embed_gather(idx, table16)
    return out.reshape(BATCH, NUM_FIELDS * EMBED_DIM)


# speedup vs baseline: 2.2987x; 1.1505x over previous
import functools

import numpy as np

import jax
import jax.numpy as jnp
from jax import lax
from jax.experimental import pallas as pl
from jax.experimental.pallas import tpu as pltpu
from jax.experimental.pallas import tpu_sc as plsc

BATCH = 16384
NUM_FIELDS = 26
VOCAB = 1000000
EMBED_DIM = 64
LANES = 16
SLICES = EMBED_DIM // LANES
J = NUM_FIELDS * SLICES

ITEMS = BATCH * J
IDX_COLS = 128
IDX_ROWS = ITEMS // IDX_COLS

NUM_CORES = 2
NUM_SUBCORES = 16
NUM_WORKERS = NUM_CORES * NUM_SUBCORES
CHUNK_B = 16
CHUNK_ROWS = CHUNK_B * J // IDX_COLS
NCHUNKS = BATCH // CHUNK_B
CHUNKS_PER_W = NCHUNKS // NUM_WORKERS
GROUPS = CHUNK_B * J // LANES

_G = np.arange(GROUPS)[:, None]
_l = np.arange(LANES)[None, :]
_sl = _G // 52
_t = (_G % 52) // 4
_rp = _G % 4
_row = _sl * 8 + 2 * _rp + _l // 8
_j = 8 * _t + (_l % 8)
_ROWT = np.ascontiguousarray(_row.astype(np.int32))
_COLT = np.ascontiguousarray((_j % NUM_FIELDS).astype(np.int32))
_ADDK = np.ascontiguousarray((_j // NUM_FIELDS).astype(np.int32))

_mesh = plsc.VectorSubcoreMesh(core_axis_name="c", subcore_axis_name="s")


@functools.partial(
    pl.kernel,
    mesh=_mesh,
    compiler_params=pltpu.CompilerParams(use_tc_tiling_on_sc=False,
                                         needs_layout_passes=False),
    out_type=jax.ShapeDtypeStruct((NCHUNKS, CHUNK_ROWS, IDX_COLS, LANES),
                                  jnp.float32),
    scratch_types=[
        pltpu.VMEM((CHUNK_B, NUM_FIELDS), jnp.int32),
        pltpu.VMEM((GROUPS, LANES), jnp.int32),
        pltpu.VMEM((GROUPS, LANES), jnp.int32),
        pltpu.VMEM((GROUPS, LANES), jnp.int32),
        pltpu.VMEM((2, CHUNK_ROWS, IDX_COLS), jnp.int32),
        pltpu.VMEM((2, CHUNK_ROWS, IDX_COLS, LANES), jnp.float32),
        pltpu.SemaphoreType.DMA,
        pltpu.SemaphoreType.DMA,
        pltpu.SemaphoreType.DMA,
        pltpu.SemaphoreType.DMA,
    ],
)
def _mh_embed_gather(x4_hbm, rowt_hbm, colt_hbm, addk_hbm, table_hbm, out_hbm,
                     xv, rowt, colt, addk, idx_v, buf_v,
                     gsem0, gsem1, osem0, osem1):
    wid = lax.axis_index("s") * NUM_CORES + lax.axis_index("c")
    gsems = (gsem0, gsem1)
    osems = (osem0, osem1)

    pltpu.sync_copy(rowt_hbm, rowt)
    pltpu.sync_copy(colt_hbm, colt)
    pltpu.sync_copy(addk_hbm, addk)

    def build(cid, slot):
        pltpu.sync_copy(x4_hbm.at[cid], xv)
        for G in range(GROUPS):
            r = rowt[G]
            c = colt[G]
            v = plsc.load_gather(xv, [r, c]) + addk[G]
            idx_v[slot, G // 8, pl.ds((G % 8) * LANES, LANES)] = v

    def fire(slot):
        for jj in range(CHUNK_ROWS):
            pltpu.async_copy(
                table_hbm.at[idx_v.at[slot, jj]], buf_v.at[slot, jj],
                gsems[slot])

    def wait_gathers(slot):
        for jj in range(CHUNK_ROWS):
            pltpu.make_async_copy(
                table_hbm.at[idx_v.at[slot, jj]], buf_v.at[slot, jj],
                gsems[slot]).wait()

    def start_write(cid, slot):
        pltpu.async_copy(buf_v.at[slot], out_hbm.at[cid], osems[slot])

    def wait_write(slot):
        pltpu.make_async_copy(
            buf_v.at[slot], out_hbm.at[0], osems[slot]).wait()

    cid0 = wid * CHUNKS_PER_W
    build(cid0, 0)

    def pair_body(p, carry):
        for b in range(2):
            g = p * 2 + b
            slot, other = b, 1 - b
            cid = cid0 + g
            @pl.when(g >= 2)
            def _():
                wait_write(slot)

            fire(slot)
            build(cid0 + jnp.minimum(g + 1, CHUNKS_PER_W - 1), other)
            wait_gathers(slot)
            start_write(cid, slot)
        return carry

    lax.fori_loop(0, CHUNKS_PER_W // 2, pair_body, 0)
    wait_write(0)
    wait_write(1)


def kernel(X, table):
    x4 = (X.astype(jnp.int32) * SLICES).reshape(NCHUNKS, CHUNK_B, NUM_FIELDS)
    table16 = table.reshape(VOCAB * SLICES, LANES)
    out = _mh_embed_gather(x4, jnp.asarray(_ROWT), jnp.asarray(_COLT),
                           jnp.asarray(_ADDK), table16)
    out = out.reshape(BATCH // 8, 13, 8, 128).transpose(0, 2, 1, 3)
    return out.reshape(BATCH, NUM_FIELDS * EMBED_DIM)


# speedup vs baseline: 2.3337x; 1.0153x over previous
import functools

import numpy as np

import jax
import jax.numpy as jnp
from jax import lax
from jax.experimental import pallas as pl
from jax.experimental.pallas import tpu as pltpu
from jax.experimental.pallas import tpu_sc as plsc

BATCH = 16384
NUM_FIELDS = 26
VOCAB = 1000000
EMBED_DIM = 64
LANES = 16
SLICES = EMBED_DIM // LANES
J = NUM_FIELDS * SLICES

ITEMS = BATCH * J
IDX_COLS = 128
IDX_ROWS = ITEMS // IDX_COLS

NUM_CORES = 2
NUM_SUBCORES = 16
NUM_WORKERS = NUM_CORES * NUM_SUBCORES
CHUNK_B = 16
CHUNK_ROWS = CHUNK_B * J // IDX_COLS
NCHUNKS = BATCH // CHUNK_B
CHUNKS_PER_W = NCHUNKS // NUM_WORKERS
GROUPS = CHUNK_B * J // LANES

_G = np.arange(GROUPS)[:, None]
_l = np.arange(LANES)[None, :]
_sl = _G // 52
_t = (_G % 52) // 4
_rp = _G % 4
_row = _sl * 8 + 2 * _rp + _l // 8
_j = 8 * _t + (_l % 8)
_ROWT = np.ascontiguousarray(_row.astype(np.int32))
_COLT = np.ascontiguousarray((_j % NUM_FIELDS).astype(np.int32))
_ADDK = np.ascontiguousarray((_j // NUM_FIELDS).astype(np.int32))

_mesh = plsc.VectorSubcoreMesh(core_axis_name="c", subcore_axis_name="s")


@functools.partial(
    pl.kernel,
    mesh=_mesh,
    compiler_params=pltpu.CompilerParams(use_tc_tiling_on_sc=False,
                                         needs_layout_passes=False),
    out_type=jax.ShapeDtypeStruct((NCHUNKS, CHUNK_ROWS, IDX_COLS, LANES),
                                  jnp.float32),
    scratch_types=[
        pltpu.VMEM((CHUNK_B, NUM_FIELDS), jnp.int32),
        pltpu.VMEM((GROUPS, LANES), jnp.int32),
        pltpu.VMEM((GROUPS, LANES), jnp.int32),
        pltpu.VMEM((GROUPS, LANES), jnp.int32),
        pltpu.VMEM((2, CHUNK_ROWS, IDX_COLS), jnp.int32),
        pltpu.VMEM((2, CHUNK_ROWS, IDX_COLS, LANES), jnp.float32),
        pltpu.SemaphoreType.DMA,
        pltpu.SemaphoreType.DMA,
        pltpu.SemaphoreType.DMA,
        pltpu.SemaphoreType.DMA,
    ],
)
def _mh_embed_gather(x4_hbm, rowt_hbm, colt_hbm, addk_hbm, table_hbm, out_hbm,
                     xv, rowt, colt, addk, idx_v, buf_v,
                     gsem0, gsem1, osem0, osem1):
    wid = lax.axis_index("s") * NUM_CORES + lax.axis_index("c")
    gsems = (gsem0, gsem1)
    osems = (osem0, osem1)

    pltpu.sync_copy(rowt_hbm, rowt)
    pltpu.sync_copy(colt_hbm, colt)
    pltpu.sync_copy(addk_hbm, addk)

    def build(cid, slot):
        pltpu.sync_copy(x4_hbm.at[cid], xv)
        for G in range(GROUPS):
            r = rowt[G]
            c = colt[G]
            v = plsc.load_gather(xv, [r, c]) + addk[G]
            idx_v[slot, G // 8, pl.ds((G % 8) * LANES, LANES)] = v

    def fire(slot):
        for jj in range(CHUNK_ROWS):
            pltpu.async_copy(
                table_hbm.at[idx_v.at[slot, jj]], buf_v.at[slot, jj],
                gsems[slot])

    def wait_gathers(slot):
        for jj in range(CHUNK_ROWS):
            pltpu.make_async_copy(
                table_hbm.at[idx_v.at[slot, jj]], buf_v.at[slot, jj],
                gsems[slot]).wait()

    def start_write(cid, slot):
        pltpu.async_copy(buf_v.at[slot], out_hbm.at[cid], osems[slot])

    def wait_write(slot):
        pltpu.make_async_copy(
            buf_v.at[slot], out_hbm.at[0], osems[slot]).wait()

    cid0 = wid * CHUNKS_PER_W
    build(cid0, 0)
    fire(0)

    def pair_body(p, carry):
        for b in range(2):
            g = p * 2 + b
            slot, other = b, 1 - b
            cid = cid0 + g
            @pl.when(g >= 1)
            def _():
                wait_write(other)

            @pl.when(g + 1 < CHUNKS_PER_W)
            def _():
                build(cid + 1, other)
                fire(other)

            wait_gathers(slot)
            start_write(cid, slot)
        return carry

    lax.fori_loop(0, CHUNKS_PER_W // 2, pair_body, 0)
    wait_write(1)


def kernel(X, table):
    x4 = (X.astype(jnp.int32) * SLICES).reshape(NCHUNKS, CHUNK_B, NUM_FIELDS)
    table16 = table.reshape(VOCAB * SLICES, LANES)
    out = _mh_embed_gather(x4, jnp.asarray(_ROWT), jnp.asarray(_COLT),
                           jnp.asarray(_ADDK), table16)
    out = out.reshape(BATCH // 8, 13, 8, 128).transpose(0, 2, 1, 3)
    return out.reshape(BATCH, NUM_FIELDS * EMBED_DIM)
